# baseline (speedup 1.0000x reference)
import functools

import jax
import jax.numpy as jnp
from jax import lax
from jax.experimental import pallas as pl
from jax.experimental.pallas import tpu as pltpu
from jax.experimental.pallas import tpu_sc as plsc

_DIM = 32
_N_EMBED = 8192
_B = 64
_T = 1024
_N_TOKENS = _B * _T
_NC = 2
_NS = 16
_NW = _NC * _NS
_BPW = _B // _NW
_G = 128
_MSE_BB = 4

_mesh = plsc.VectorSubcoreMesh(core_axis_name="c", subcore_axis_name="s")


@functools.partial(
    pl.kernel,
    out_type=jax.ShapeDtypeStruct((_B, _DIM // 8, _T // 128, 8, 128),
                                  jnp.float32),
    mesh=_mesh,
    scratch_types=[
        pltpu.VMEM((_BPW * _T,), jnp.int32),
        pltpu.VMEM((_T // _G, _G, _DIM), jnp.float32),
        pltpu.VMEM((_T // _G, _G, _DIM), jnp.float32),
        pltpu.SemaphoreType.DMA,
        pltpu.SemaphoreType.DMA,
        pltpu.SemaphoreType.DMA,
        pltpu.SemaphoreType.DMA,
    ],
    compiler_params=pltpu.CompilerParams(use_tc_tiling_on_sc=False),
)
def _vq_gather(lab_hbm, emb_hbm, q_hbm,
               idx_v, rows0, rows1, g0, g1, s0, s1):
    wid = lax.axis_index("s") * _NC + lax.axis_index("c")
    b0 = wid * _BPW
    pltpu.sync_copy(lab_hbm.at[pl.ds(b0 * _T, _BPW * _T)], idx_v)

    bufs = [rows0, rows1]
    gsems = [g0, g1]
    osems = [s0, s1]

    def fire(k):
        buf, sem = bufs[k % 2], gsems[k % 2]
        return [
            pltpu.async_copy(
                emb_hbm.at[idx_v.at[pl.ds(k * _T + j * _G, _G)]],
                buf.at[j],
                sem,
            )
            for j in range(_T // _G)
        ]

    gathers = {0: fire(0)}
    stores = {}
    for k in range(_BPW):
        if k + 1 < _BPW:
            gathers[k + 1] = fire(k + 1)
        for cp in gathers[k]:
            cp.wait()
        buf = bufs[k % 2]
        stores[k] = [
            pltpu.async_copy(
                buf.at[:, :, d],
                q_hbm.at[b0 + k, d // 8, :, d % 8, :],
                osems[k % 2],
            )
            for d in range(_DIM)
        ]
    for k in range(_BPW):
        for cp in stores[k]:
            cp.wait()


@functools.partial(
    pl.pallas_call,
    grid=(_B // _MSE_BB,),
    in_specs=[
        pl.BlockSpec((_MSE_BB, _DIM, _T), lambda i: (i, 0, 0)),
        pl.BlockSpec((_MSE_BB, _DIM, _T), lambda i: (i, 0, 0)),
    ],
    out_specs=[
        pl.BlockSpec((1, 1, 128), lambda i: (i, 0, 0)),
    ],
    out_shape=[
        jax.ShapeDtypeStruct((_B // _MSE_BB, 1, 128), jnp.float32),
    ],
)
def _mse(q_ref, x_ref, p_ref):
    d = q_ref[...] - x_ref[...]
    p_ref[...] = jnp.broadcast_to(jnp.sum(d * d), (1, 1, 128))


def kernel(input, labels, embed):
    emb_t = embed.T
    q4 = _vq_gather(labels, emb_t)
    q3 = q4.transpose(0, 1, 3, 2, 4).reshape(_B, _DIM, _T)
    x_t = input.transpose(0, 2, 1)
    partials, = _mse(q3, x_t)
    quantize = q3.transpose(0, 2, 1)
    diff = jnp.sum(partials[:, 0, 0]) / jnp.float32(_N_TOKENS * _DIM)
    embed_ind = labels.reshape(_B, _T)
    return quantize, diff, embed_ind


# speedup vs baseline: 89.8658x; 89.8658x over previous
import functools

import jax
import jax.numpy as jnp
from jax import lax
from jax.experimental import pallas as pl
from jax.experimental.pallas import tpu as pltpu
from jax.experimental.pallas import tpu_sc as plsc

_DIM = 32
_N_EMBED = 8192
_B = 64
_T = 1024
_N_TOKENS = _B * _T
_NC = 2
_NS = 16
_NW = _NC * _NS
_DG = 4
_DPG = _DIM // _DG
_BG = _NW // _DG
_BPG = _B // _BG
_MSE_BB = 4

_mesh = plsc.VectorSubcoreMesh(core_axis_name="c", subcore_axis_name="s")


@functools.partial(
    pl.kernel,
    out_type=jax.ShapeDtypeStruct((_B, _DG, _T // 128, _DPG, 128),
                                  jnp.float32),
    mesh=_mesh,
    scratch_types=[
        pltpu.VMEM((_DPG, _N_EMBED), jnp.float32),
        pltpu.VMEM((_T,), jnp.int32),
        pltpu.VMEM((_T,), jnp.int32),
        pltpu.VMEM((_T // 128, _DPG, 128), jnp.float32),
        pltpu.VMEM((_T // 128, _DPG, 128), jnp.float32),
        pltpu.SemaphoreType.DMA,
        pltpu.SemaphoreType.DMA,
        pltpu.SemaphoreType.DMA,
        pltpu.SemaphoreType.DMA,
        pltpu.SemaphoreType.DMA,
    ],
    compiler_params=pltpu.CompilerParams(use_tc_tiling_on_sc=False,
                                         needs_layout_passes=False),
)
def _vq_lookup(lab_hbm, emb_hbm, q_hbm,
               tab_v, idx0, idx1, out0, out1, tsem, i0, i1, o0, o1):
    wid = lax.axis_index("s") * _NC + lax.axis_index("c")
    bg = wid // _DG
    dg = wid % _DG
    b0 = bg * _BPG

    tab_cp = pltpu.async_copy(emb_hbm.at[pl.ds(dg * _DPG, _DPG), :],
                              tab_v, tsem)
    idxs = [idx0, idx1]
    isems = [i0, i1]
    outs = [out0, out1]
    osems = [o0, o1]
    dl_ids = [jnp.full((16,), dl, jnp.int32) for dl in range(_DPG)]

    def load_idx(k):
        return pltpu.async_copy(
            lab_hbm.at[pl.ds((b0 + k) * _T, _T)], idxs[k % 2], isems[k % 2])

    icopies = {0: load_idx(0)}
    ocopies = {}
    tab_cp.wait()
    for k in range(_BPG):
        if k + 1 < _BPG:
            icopies[k + 1] = load_idx(k + 1)
        icopies[k].wait()
        if k - 2 >= 0:
            ocopies[k - 2].wait()
        idx_v = idxs[k % 2]
        out_v = outs[k % 2]

        @pl.loop(0, _T, step=32)
        def _(t0):
            j = t0 // 128
            tl = t0 % 128
            iva = idx_v[pl.ds(t0, 16)]
            ivb = idx_v[pl.ds(t0 + 16, 16)]
            vals = [plsc.load_gather(tab_v, [dl_ids[dl], iva])
                    for dl in range(_DPG)]
            vals += [plsc.load_gather(tab_v, [dl_ids[dl], ivb])
                     for dl in range(_DPG)]
            for dl in range(_DPG):
                out_v[j, dl, pl.ds(tl, 16)] = vals[dl]
                out_v[j, dl, pl.ds(tl + 16, 16)] = vals[_DPG + dl]

        ocopies[k] = pltpu.async_copy(
            out_v, q_hbm.at[b0 + k, dg], osems[k % 2])
    ocopies[_BPG - 2].wait()
    ocopies[_BPG - 1].wait()


@functools.partial(
    pl.pallas_call,
    grid=(_B // _MSE_BB,),
    in_specs=[
        pl.BlockSpec((_MSE_BB, _DIM, _T), lambda i: (i, 0, 0)),
        pl.BlockSpec((_MSE_BB, _DIM, _T), lambda i: (i, 0, 0)),
    ],
    out_specs=[
        pl.BlockSpec((1, 1, 128), lambda i: (i, 0, 0)),
    ],
    out_shape=[
        jax.ShapeDtypeStruct((_B // _MSE_BB, 1, 128), jnp.float32),
    ],
)
def _mse(q_ref, x_ref, p_ref):
    d = q_ref[...] - x_ref[...]
    p_ref[...] = jnp.broadcast_to(jnp.sum(d * d), (1, 1, 128))


def kernel(input, labels, embed):
    q4 = _vq_lookup(labels, embed)
    q3 = q4.transpose(0, 1, 3, 2, 4).reshape(_B, _DIM, _T)
    x_t = input.transpose(0, 2, 1)
    partials, = _mse(q3, x_t)
    quantize = q3.transpose(0, 2, 1)
    diff = jnp.sum(partials[:, 0, 0]) / jnp.float32(_N_TOKENS * _DIM)
    embed_ind = labels.reshape(_B, _T)
    return quantize, diff, embed_ind


# speedup vs baseline: 98.1857x; 1.0926x over previous
import functools

import jax
import jax.numpy as jnp
from jax import lax
from jax.experimental import pallas as pl
from jax.experimental.pallas import tpu as pltpu
from jax.experimental.pallas import tpu_sc as plsc

_DIM = 32
_N_EMBED = 8192
_B = 64
_T = 1024
_N_TOKENS = _B * _T
_NC = 2
_NS = 16
_NW = _NC * _NS
_DG = 4
_DPG = _DIM // _DG
_BG = _NW // _DG
_BPG = _B // _BG
_MSE_BB = 8

_mesh = plsc.VectorSubcoreMesh(core_axis_name="c", subcore_axis_name="s")


@functools.partial(
    pl.kernel,
    out_type=jax.ShapeDtypeStruct((_B, _DG, _T // 128, _DPG, 128),
                                  jnp.float32),
    mesh=_mesh,
    scratch_types=[
        pltpu.VMEM((_DPG, _N_EMBED), jnp.float32),
        pltpu.VMEM((_T,), jnp.int32),
        pltpu.VMEM((_T,), jnp.int32),
        pltpu.VMEM((_T // 128, _DPG, 128), jnp.float32),
        pltpu.VMEM((_T // 128, _DPG, 128), jnp.float32),
        pltpu.SemaphoreType.DMA,
        pltpu.SemaphoreType.DMA,
        pltpu.SemaphoreType.DMA,
        pltpu.SemaphoreType.DMA,
        pltpu.SemaphoreType.DMA,
    ],
    compiler_params=pltpu.CompilerParams(use_tc_tiling_on_sc=False,
                                         needs_layout_passes=False),
)
def _vq_lookup(lab_hbm, emb_hbm, q_hbm,
               tab_v, idx0, idx1, out0, out1, tsem, i0, i1, o0, o1):
    wid = lax.axis_index("s") * _NC + lax.axis_index("c")
    bg = wid // _DG
    dg = wid % _DG
    b0 = bg * _BPG

    tab_cp = pltpu.async_copy(emb_hbm.at[pl.ds(dg * _DPG, _DPG), :],
                              tab_v, tsem)
    idxs = [idx0, idx1]
    isems = [i0, i1]
    outs = [out0, out1]
    osems = [o0, o1]
    dl_ids = [jnp.full((16,), dl, jnp.int32) for dl in range(_DPG)]

    def load_idx(k):
        return pltpu.async_copy(
            lab_hbm.at[pl.ds((b0 + k) * _T, _T)], idxs[k % 2], isems[k % 2])

    icopies = {0: load_idx(0)}
    ocopies = {}
    tab_cp.wait()
    for k in range(_BPG):
        if k + 1 < _BPG:
            icopies[k + 1] = load_idx(k + 1)
        icopies[k].wait()
        if k - 2 >= 0:
            ocopies[k - 2].wait()
        idx_v = idxs[k % 2]
        out_v = outs[k % 2]

        @pl.loop(0, _T, step=32)
        def _(t0):
            j = t0 // 128
            tl = t0 % 128
            iva = idx_v[pl.ds(t0, 16)]
            ivb = idx_v[pl.ds(t0 + 16, 16)]
            vals = [plsc.load_gather(tab_v, [dl_ids[dl], iva])
                    for dl in range(_DPG)]
            vals += [plsc.load_gather(tab_v, [dl_ids[dl], ivb])
                     for dl in range(_DPG)]
            for dl in range(_DPG):
                out_v[j, dl, pl.ds(tl, 16)] = vals[dl]
                out_v[j, dl, pl.ds(tl + 16, 16)] = vals[_DPG + dl]

        ocopies[k] = pltpu.async_copy(
            out_v, q_hbm.at[b0 + k, dg], osems[k % 2])
    ocopies[_BPG - 2].wait()
    ocopies[_BPG - 1].wait()


@functools.partial(
    pl.pallas_call,
    grid=(_B // _MSE_BB,),
    in_specs=[
        pl.BlockSpec((_MSE_BB, _DIM, _T), lambda i: (i, 0, 0)),
        pl.BlockSpec((_MSE_BB, _DIM, _T), lambda i: (i, 0, 0)),
    ],
    out_specs=[
        pl.BlockSpec((1, 1, _T), lambda i: (i, 0, 0)),
    ],
    out_shape=[
        jax.ShapeDtypeStruct((_B // _MSE_BB, 1, _T), jnp.float32),
    ],
)
def _mse(q_ref, x_ref, p_ref):
    d = q_ref[...] - x_ref[...]
    p_ref[...] = jnp.sum(d * d, axis=(0, 1)).reshape(1, 1, _T)


def kernel(input, labels, embed):
    q4 = _vq_lookup(labels, embed)
    q3 = q4.transpose(0, 1, 3, 2, 4).reshape(_B, _DIM, _T)
    x_t = input.transpose(0, 2, 1)
    partials, = _mse(q3, x_t)
    quantize = q3.transpose(0, 2, 1)
    diff = jnp.sum(partials) / jnp.float32(_N_TOKENS * _DIM)
    embed_ind = labels.reshape(_B, _T)
    return quantize, diff, embed_ind


# speedup vs baseline: 100.3853x; 1.0224x over previous
import functools

import jax
import jax.numpy as jnp
from jax import lax
from jax.experimental import pallas as pl
from jax.experimental.pallas import tpu as pltpu
from jax.experimental.pallas import tpu_sc as plsc

_DIM = 32
_N_EMBED = 8192
_B = 64
_T = 1024
_N_TOKENS = _B * _T
_NC = 2
_NS = 16
_NW = _NC * _NS
_DG = 4
_DPG = _DIM // _DG
_BG = _NW // _DG
_BPG = _B // _BG
_MSE_BB = 16

_mesh = plsc.VectorSubcoreMesh(core_axis_name="c", subcore_axis_name="s")


@functools.partial(
    pl.kernel,
    out_type=jax.ShapeDtypeStruct((_B, _DG, _T // 128, _DPG, 128),
                                  jnp.float32),
    mesh=_mesh,
    scratch_types=[
        pltpu.VMEM((_DPG * _N_EMBED // 128, 128), jnp.float32),
        pltpu.VMEM((_T,), jnp.int32),
        pltpu.VMEM((_T,), jnp.int32),
        pltpu.VMEM((_T // 128, _DPG, 128), jnp.float32),
        pltpu.VMEM((_T // 128, _DPG, 128), jnp.float32),
        pltpu.SemaphoreType.DMA,
        pltpu.SemaphoreType.DMA,
        pltpu.SemaphoreType.DMA,
        pltpu.SemaphoreType.DMA,
        pltpu.SemaphoreType.DMA,
    ],
    compiler_params=pltpu.CompilerParams(use_tc_tiling_on_sc=False,
                                         needs_layout_passes=False),
)
def _vq_lookup(lab_hbm, emb_hbm, q_hbm,
               tab_v, idx0, idx1, out0, out1, tsem, i0, i1, o0, o1):
    wid = lax.axis_index("s") * _NC + lax.axis_index("c")
    bg = wid // _DG
    dg = wid % _DG
    b0 = bg * _BPG

    tab_cp = pltpu.async_copy(
        emb_hbm.at[pl.ds(dg * (_DPG * _N_EMBED // 128), _DPG * _N_EMBED // 128)],
        tab_v, tsem)
    idxs = [idx0, idx1]
    isems = [i0, i1]
    outs = [out0, out1]
    osems = [o0, o1]
    dl_off = [jnp.full((16,), dl, jnp.int32) for dl in range(_DPG)]

    def load_idx(k):
        return pltpu.async_copy(
            lab_hbm.at[pl.ds((b0 + k) * _T, _T)], idxs[k % 2], isems[k % 2])

    icopies = {0: load_idx(0)}
    ocopies = {}
    tab_cp.wait()
    for k in range(_BPG):
        if k + 1 < _BPG:
            icopies[k + 1] = load_idx(k + 1)
        icopies[k].wait()
        if k - 2 >= 0:
            ocopies[k - 2].wait()
        idx_v = idxs[k % 2]
        out_v = outs[k % 2]

        @pl.loop(0, _T, step=64)
        def _(t0):
            j = t0 // 128
            tl = t0 % 128
            ivs = [idx_v[pl.ds(t0 + 16 * u, 16)] for u in range(4)]
            rows = [(iv >> 7) * 8 for iv in ivs]
            cols = [iv & 127 for iv in ivs]
            vals = [plsc.load_gather(tab_v, [rows[u] + dl_off[dl], cols[u]])
                    for u in range(4) for dl in range(_DPG)]
            for u in range(4):
                for dl in range(_DPG):
                    out_v[j, dl, pl.ds(tl + 16 * u, 16)] = vals[u * _DPG + dl]

        ocopies[k] = pltpu.async_copy(
            out_v, q_hbm.at[b0 + k, dg], osems[k % 2])
    ocopies[_BPG - 2].wait()
    ocopies[_BPG - 1].wait()


@functools.partial(
    pl.pallas_call,
    grid=(_B // _MSE_BB,),
    in_specs=[
        pl.BlockSpec((_MSE_BB, _DIM, _T), lambda i: (i, 0, 0)),
        pl.BlockSpec((_MSE_BB, _DIM, _T), lambda i: (i, 0, 0)),
    ],
    out_specs=[
        pl.BlockSpec((1, 1, _T), lambda i: (i, 0, 0)),
    ],
    out_shape=[
        jax.ShapeDtypeStruct((_B // _MSE_BB, 1, _T), jnp.float32),
    ],
)
def _mse(q_ref, x_ref, p_ref):
    d = q_ref[...] - x_ref[...]
    p_ref[...] = jnp.sum(d * d, axis=(0, 1)).reshape(1, 1, _T)


def kernel(input, labels, embed):
    emb_tiles = (embed.reshape(_DG, _DPG, _N_EMBED // 128, 128)
                 .transpose(0, 2, 1, 3)
                 .reshape(_DIM * _N_EMBED // 128, 128))
    q4 = _vq_lookup(labels, emb_tiles)
    q3 = q4.transpose(0, 1, 3, 2, 4).reshape(_B, _DIM, _T)
    x_t = input.transpose(0, 2, 1)
    partials, = _mse(q3, x_t)
    quantize = q3.transpose(0, 2, 1)
    diff = jnp.sum(partials) / jnp.float32(_N_TOKENS * _DIM)
    embed_ind = labels.reshape(_B, _T)
    return quantize, diff, embed_ind


# speedup vs baseline: 109.1695x; 1.0875x over previous
import functools

import jax
import jax.numpy as jnp
from jax import lax
from jax.experimental import pallas as pl
from jax.experimental.pallas import tpu as pltpu
from jax.experimental.pallas import tpu_sc as plsc

_DIM = 32
_N_EMBED = 8192
_B = 64
_T = 1024
_N_TOKENS = _B * _T
_NC = 2
_NS = 16
_NW = _NC * _NS
_DG = 4
_DPG = _DIM // _DG
_BG = _NW // _DG
_BPG = _B // _BG
_MSE_BB = 16

_mesh = plsc.VectorSubcoreMesh(core_axis_name="c", subcore_axis_name="s")


@functools.partial(
    pl.kernel,
    out_type=(
        jax.ShapeDtypeStruct((_B, _DG, _T // 128, _DPG, 128), jnp.float32),
        jax.ShapeDtypeStruct((_NW * 16,), jnp.float32),
    ),
    mesh=_mesh,
    scratch_types=[
        pltpu.VMEM((_DPG * _N_EMBED // 128, 128), jnp.float32),
        pltpu.VMEM((_T,), jnp.int32),
        pltpu.VMEM((_T,), jnp.int32),
        pltpu.VMEM((_T // 128, _DPG, 128), jnp.float32),
        pltpu.VMEM((_T // 128, _DPG, 128), jnp.float32),
        pltpu.VMEM((_T // 128, _DPG, 128), jnp.float32),
        pltpu.VMEM((_T // 128, _DPG, 128), jnp.float32),
        pltpu.VMEM((16,), jnp.float32),
        pltpu.SemaphoreType.DMA,
        pltpu.SemaphoreType.DMA,
        pltpu.SemaphoreType.DMA,
        pltpu.SemaphoreType.DMA,
        pltpu.SemaphoreType.DMA,
        pltpu.SemaphoreType.DMA,
        pltpu.SemaphoreType.DMA,
    ],
    compiler_params=pltpu.CompilerParams(use_tc_tiling_on_sc=False,
                                         needs_layout_passes=False),
)
def _vq_lookup(lab_hbm, emb_hbm, x_hbm, q_hbm, part_hbm,
               tab_v, idx0, idx1, out0, out1, xin0, xin1, acc_v,
               tsem, i0, i1, o0, o1, x0, x1):
    wid = lax.axis_index("s") * _NC + lax.axis_index("c")
    bg = wid // _DG
    dg = wid % _DG
    b0 = bg * _BPG

    tab_cp = pltpu.async_copy(
        emb_hbm.at[pl.ds(dg * (_DPG * _N_EMBED // 128), _DPG * _N_EMBED // 128)],
        tab_v, tsem)
    idxs = [idx0, idx1]
    isems = [i0, i1]
    outs = [out0, out1]
    osems = [o0, o1]
    xins = [xin0, xin1]
    xsems = [x0, x1]
    dl_off = [jnp.full((16,), dl, jnp.int32) for dl in range(_DPG)]
    acc_v[...] = jnp.zeros((16,), jnp.float32)

    def load_idx(k):
        return pltpu.async_copy(
            lab_hbm.at[pl.ds((b0 + k) * _T, _T)], idxs[k % 2], isems[k % 2])

    def load_x(k):
        return pltpu.async_copy(
            x_hbm.at[b0 + k, dg], xins[k % 2], xsems[k % 2])

    icopies = {0: load_idx(0)}
    xcopies = {0: load_x(0)}
    ocopies = {}
    tab_cp.wait()
    for k in range(_BPG):
        if k + 1 < _BPG:
            icopies[k + 1] = load_idx(k + 1)
            xcopies[k + 1] = load_x(k + 1)
        icopies[k].wait()
        xcopies[k].wait()
        if k - 2 >= 0:
            ocopies[k - 2].wait()
        idx_v = idxs[k % 2]
        out_v = outs[k % 2]
        xin_v = xins[k % 2]

        @pl.loop(0, _T, step=64)
        def _(t0):
            j = t0 // 128
            tl = t0 % 128
            ivs = [idx_v[pl.ds(t0 + 16 * u, 16)] for u in range(4)]
            rows = [(iv >> 7) * 8 for iv in ivs]
            cols = [iv & 127 for iv in ivs]
            vals = [plsc.load_gather(tab_v, [rows[u] + dl_off[dl], cols[u]])
                    for u in range(4) for dl in range(_DPG)]
            sqs = []
            for u in range(4):
                sq = None
                for dl in range(_DPG):
                    v = vals[u * _DPG + dl]
                    out_v[j, dl, pl.ds(tl + 16 * u, 16)] = v
                    dd = v - xin_v[j, dl, pl.ds(tl + 16 * u, 16)]
                    sq = dd * dd if sq is None else sq + dd * dd
                sqs.append(sq)
            acc_v[...] = acc_v[...] + ((sqs[0] + sqs[1]) + (sqs[2] + sqs[3]))

        ocopies[k] = pltpu.async_copy(
            out_v, q_hbm.at[b0 + k, dg], osems[k % 2])
    pltpu.sync_copy(acc_v, part_hbm.at[pl.ds(wid * 16, 16)])
    ocopies[_BPG - 2].wait()
    ocopies[_BPG - 1].wait()


@functools.partial(
    pl.pallas_call,
    grid=(_B // _MSE_BB,),
    in_specs=[
        pl.BlockSpec((_MSE_BB, _DIM, _T), lambda i: (i, 0, 0)),
        pl.BlockSpec((_MSE_BB, _DIM, _T), lambda i: (i, 0, 0)),
    ],
    out_specs=[
        pl.BlockSpec((1, 1, _T), lambda i: (i, 0, 0)),
    ],
    out_shape=[
        jax.ShapeDtypeStruct((_B // _MSE_BB, 1, _T), jnp.float32),
    ],
)
def _mse(q_ref, x_ref, p_ref):
    d = q_ref[...] - x_ref[...]
    p_ref[...] = jnp.sum(d * d, axis=(0, 1)).reshape(1, 1, _T)


def kernel(input, labels, embed):
    emb_tiles = (embed.reshape(_DG, _DPG, _N_EMBED // 128, 128)
                 .transpose(0, 2, 1, 3)
                 .reshape(_DIM * _N_EMBED // 128, 128))
    x4 = (input.transpose(0, 2, 1)
          .reshape(_B, _DG, _DPG, _T // 128, 128)
          .transpose(0, 1, 3, 2, 4))
    q4, partials = _vq_lookup(labels, emb_tiles, x4)
    q3 = q4.transpose(0, 1, 3, 2, 4).reshape(_B, _DIM, _T)
    quantize = q3.transpose(0, 2, 1)
    diff = jnp.sum(partials) / jnp.float32(_N_TOKENS * _DIM)
    embed_ind = labels.reshape(_B, _T)
    return quantize, diff, embed_ind


# speedup vs baseline: 109.7205x; 1.0050x over previous
import functools

import jax
import jax.numpy as jnp
from jax import lax
from jax.experimental import pallas as pl
from jax.experimental.pallas import tpu as pltpu
from jax.experimental.pallas import tpu_sc as plsc

_DIM = 32
_N_EMBED = 8192
_B = 64
_T = 1024
_N_TOKENS = _B * _T
_NC = 2
_NS = 16
_NW = _NC * _NS
_DG = 4
_DPG = _DIM // _DG
_BG = _NW // _DG
_BPG = _B // _BG

_mesh = plsc.VectorSubcoreMesh(core_axis_name="c", subcore_axis_name="s")


@functools.partial(
    pl.kernel,
    out_type=(
        jax.ShapeDtypeStruct((_B, _DG, _T // 128, _DPG, 128), jnp.float32),
        jax.ShapeDtypeStruct((_NW * 16,), jnp.float32),
    ),
    mesh=_mesh,
    scratch_types=[
        pltpu.VMEM((_DPG * _N_EMBED // 128, 128), jnp.float32),
        pltpu.VMEM((_T,), jnp.int32),
        pltpu.VMEM((_T,), jnp.int32),
        pltpu.VMEM((_T // 128, _DPG, 128), jnp.float32),
        pltpu.VMEM((_T // 128, _DPG, 128), jnp.float32),
        pltpu.VMEM((_T // 128, _DPG, 128), jnp.float32),
        pltpu.VMEM((_T // 128, _DPG, 128), jnp.float32),
        pltpu.VMEM((16,), jnp.float32),
        pltpu.SemaphoreType.DMA,
        pltpu.SemaphoreType.DMA,
        pltpu.SemaphoreType.DMA,
        pltpu.SemaphoreType.DMA,
        pltpu.SemaphoreType.DMA,
        pltpu.SemaphoreType.DMA,
        pltpu.SemaphoreType.DMA,
    ],
    compiler_params=pltpu.CompilerParams(use_tc_tiling_on_sc=False,
                                         needs_layout_passes=False),
)
def _vq_lookup(lab_hbm, emb_hbm, x_hbm, q_hbm, part_hbm,
               tab_v, idx0, idx1, out0, out1, xin0, xin1, acc_v,
               tsem, i0, i1, o0, o1, x0, x1):
    wid = lax.axis_index("s") * _NC + lax.axis_index("c")
    bg = wid // _DG
    dg = wid % _DG
    b0 = bg * _BPG

    tab_cp = pltpu.async_copy(
        emb_hbm.at[pl.ds(dg * (_DPG * _N_EMBED // 128), _DPG * _N_EMBED // 128)],
        tab_v, tsem)
    idxs = [idx0, idx1]
    isems = [i0, i1]
    outs = [out0, out1]
    osems = [o0, o1]
    xins = [xin0, xin1]
    xsems = [x0, x1]
    dl_off = [jnp.full((16,), dl, jnp.int32) for dl in range(_DPG)]
    acc_v[...] = jnp.zeros((16,), jnp.float32)

    def load_idx(k):
        return pltpu.async_copy(
            lab_hbm.at[pl.ds((b0 + k) * _T, _T)], idxs[k % 2], isems[k % 2])

    def load_x(k):
        return pltpu.async_copy(
            x_hbm.at[b0 + k, dg], xins[k % 2], xsems[k % 2])

    icopies = {0: load_idx(0)}
    xcopies = {0: load_x(0)}
    ocopies = {}
    tab_cp.wait()
    for k in range(_BPG):
        if k + 1 < _BPG:
            icopies[k + 1] = load_idx(k + 1)
            xcopies[k + 1] = load_x(k + 1)
        icopies[k].wait()
        xcopies[k].wait()
        if k - 2 >= 0:
            ocopies[k - 2].wait()
        idx_v = idxs[k % 2]
        out_v = outs[k % 2]
        xin_v = xins[k % 2]

        @pl.loop(0, _T, step=64)
        def _(t0):
            j = t0 // 128
            tl = t0 % 128
            ivs = [idx_v[pl.ds(t0 + 16 * u, 16)] for u in range(4)]
            rows = [(iv >> 7) * 8 for iv in ivs]
            cols = [iv & 127 for iv in ivs]
            vals = [plsc.load_gather(tab_v, [rows[u] + dl_off[dl], cols[u]])
                    for u in range(4) for dl in range(_DPG)]
            sqs = []
            for u in range(4):
                sq = None
                for dl in range(_DPG):
                    v = vals[u * _DPG + dl]
                    out_v[j, dl, pl.ds(tl + 16 * u, 16)] = v
                    dd = v - xin_v[j, dl, pl.ds(tl + 16 * u, 16)]
                    sq = dd * dd if sq is None else sq + dd * dd
                sqs.append(sq)
            acc_v[...] = acc_v[...] + ((sqs[0] + sqs[1]) + (sqs[2] + sqs[3]))

        ocopies[k] = pltpu.async_copy(
            out_v, q_hbm.at[b0 + k, dg], osems[k % 2])
    pltpu.sync_copy(acc_v, part_hbm.at[pl.ds(wid * 16, 16)])
    ocopies[_BPG - 2].wait()
    ocopies[_BPG - 1].wait()


def kernel(input, labels, embed):
    emb_tiles = (embed.reshape(_DG, _DPG, _N_EMBED // 128, 128)
                 .transpose(0, 2, 1, 3)
                 .reshape(_DIM * _N_EMBED // 128, 128))
    x4 = (input.transpose(0, 2, 1)
          .reshape(_B, _DG, _DPG, _T // 128, 128)
          .transpose(0, 1, 3, 2, 4))
    q4, partials = _vq_lookup(labels, emb_tiles, x4)
    q3 = q4.transpose(0, 1, 3, 2, 4).reshape(_B, _DIM, _T)
    quantize = q3.transpose(0, 2, 1)
    diff = jnp.sum(partials) / jnp.float32(_N_TOKENS * _DIM)
    embed_ind = labels.reshape(_B, _T)
    return quantize, diff, embed_ind
